# baseline (speedup 1.0000x reference)
import jax
import jax.numpy as jnp
from jax.experimental import pallas as pl
from jax.experimental.pallas import tpu as pltpu
from jax.experimental.pallas import tpu_sc as plsc

VOCAB = 100000
EMBED = 128
SEQ = 20
HIDDEN = 2048
OUT = 1000
BATCH = 16384

CHUNKS = (2048, 4096, 5120, 5120)

WINDOW = 128
BM = 512


def _sc_gather(table, idx_all, n_idx, idx_off):
    mesh = plsc.VectorSubcoreMesh(core_axis_name="core",
                                  subcore_axis_name="subcore")
    off_w = idx_off // WINDOW

    @pl.kernel(
        out_type=jax.ShapeDtypeStruct((n_idx, EMBED), table.dtype),
        mesh=mesh,
    )
    def gather_kernel(tab_hbm, i_hbm, o_hbm):
        def body(i_vmem, o_vmem):
            pltpu.sync_copy(tab_hbm.at[i_vmem.at[0]], o_vmem)

        pltpu.emit_pipeline(
            body,
            grid=(n_idx // WINDOW,),
            in_specs=[pl.BlockSpec((1, WINDOW),
                                   index_map=lambda i: (0, off_w + i))],
            out_specs=[pl.BlockSpec((WINDOW, EMBED),
                                    index_map=lambda i: (i, 0))],
            core_axis_name=("core", "subcore"),
            dimension_semantics=(pltpu.PARALLEL,),
        )(i_hbm, o_hbm)

    return gather_kernel(table, idx_all)


def _mlp_body(flat_ref, w1_ref, b1_ref, w2t_ref, b2_ref, out_ref):
    flat = flat_ref.reshape(BM, SEQ * EMBED)[...].astype(jnp.bfloat16)
    h = jnp.dot(flat, w1_ref[...], preferred_element_type=jnp.float32)
    h = jnp.maximum(h + b1_ref[...], 0.0).astype(jnp.bfloat16)
    out_t = jax.lax.dot_general(
        w2t_ref[...], h, (((1,), (1,)), ((), ())),
        preferred_element_type=jnp.float32,
    )
    out_ref[...] = out_t + b2_ref[...]


def _mlp_body_aliased(flat_ref, w1_ref, b1_ref, w2t_ref, b2_ref, prev_ref,
                      out_ref):
    del prev_ref
    _mlp_body(flat_ref, w1_ref, b1_ref, w2t_ref, b2_ref, out_ref)


def _mlp_chunk(flat, W1, b1, W2T, b2, prev, cb, row0):
    base = row0 // BM
    in_specs = [
        pl.BlockSpec((BM * SEQ, EMBED), lambda i: (i, 0)),
        pl.BlockSpec((SEQ * EMBED, HIDDEN), lambda i: (0, 0)),
        pl.BlockSpec((1, HIDDEN), lambda i: (0, 0)),
        pl.BlockSpec((OUT, HIDDEN), lambda i: (0, 0)),
        pl.BlockSpec((OUT, 1), lambda i: (0, 0)),
    ]
    args = [flat, W1, b1, W2T, b2]
    body = _mlp_body
    aliases = {}
    if prev is not None:
        in_specs.append(pl.BlockSpec(memory_space=pl.ANY))
        args.append(prev)
        body = _mlp_body_aliased
        aliases = {5: 0}
    return pl.pallas_call(
        body,
        grid=(cb // BM,),
        in_specs=in_specs,
        out_specs=pl.BlockSpec((OUT, BM), lambda i: (0, base + i)),
        out_shape=jax.ShapeDtypeStruct((OUT, BATCH), jnp.float32),
        input_output_aliases=aliases,
    )(*args)


def kernel(x, table, W1, b1, W2, b2):
    w1_h = W1.astype(jnp.bfloat16)
    w2t_h = W2.T.astype(jnp.bfloat16)
    b1r = b1.reshape(1, HIDDEN)
    b2r = b2.reshape(OUT, 1)
    starts = [sum(CHUNKS[:c]) for c in range(len(CHUNKS))]
    idx0 = x[:CHUNKS[0]].reshape(1, CHUNKS[0] * SEQ)
    idx_all = x.reshape(1, BATCH * SEQ)
    flats = [_sc_gather(table, idx0, CHUNKS[0] * SEQ, 0)]
    flats += [_sc_gather(table, idx_all, cb * SEQ, r0 * SEQ)
              for r0, cb in zip(starts[1:], CHUNKS[1:])]
    out_t = None
    for c, (r0, cb) in enumerate(zip(starts, CHUNKS)):
        out_t = _mlp_chunk(flats[c], w1_h, b1r, w2t_h, b2r, out_t, cb, r0)
    return out_t.T


# speedup vs baseline: 1.0538x; 1.0538x over previous
import jax
import jax.numpy as jnp
from jax.experimental import pallas as pl
from jax.experimental.pallas import tpu as pltpu
from jax.experimental.pallas import tpu_sc as plsc

VOCAB = 100000
EMBED = 128
SEQ = 20
HIDDEN = 2048
OUT = 1000
BATCH = 16384

CHUNKS = (2048, 4096, 5120, 5120)

WINDOW = 128
BM = 512


def _sc_gather(table, idx_all, n_idx, idx_off):
    mesh = plsc.VectorSubcoreMesh(core_axis_name="core",
                                  subcore_axis_name="subcore")
    off_w = idx_off // WINDOW

    @pl.kernel(
        out_type=jax.ShapeDtypeStruct((n_idx, EMBED), table.dtype),
        mesh=mesh,
    )
    def gather_kernel(tab_hbm, i_hbm, o_hbm):
        def body(i_vmem, o_vmem):
            pltpu.sync_copy(tab_hbm.at[i_vmem.at[0]], o_vmem)

        pltpu.emit_pipeline(
            body,
            grid=(n_idx // WINDOW,),
            in_specs=[pl.BlockSpec((1, WINDOW),
                                   index_map=lambda i: (0, off_w + i))],
            out_specs=[pl.BlockSpec((WINDOW, EMBED),
                                    index_map=lambda i: (i, 0))],
            core_axis_name=("core", "subcore"),
            dimension_semantics=(pltpu.PARALLEL,),
        )(i_hbm, o_hbm)

    return gather_kernel(table, idx_all)


def _mlp_body(flat_ref, w1_ref, b1_ref, w2t_ref, b2_ref, out_ref):
    flat = flat_ref.reshape(BM, SEQ * EMBED)[...].astype(jnp.bfloat16)
    h = jnp.dot(flat, w1_ref[...], preferred_element_type=jnp.float32)
    h = jnp.maximum(h + b1_ref[...], 0.0).astype(jnp.bfloat16)
    out_t = jax.lax.dot_general(
        w2t_ref[...], h, (((1,), (1,)), ((), ())),
        preferred_element_type=jnp.float32,
    )
    out_ref[...] = out_t + b2_ref[...]


def _mlp_body_aliased(flat_ref, w1_ref, b1_ref, w2t_ref, b2_ref, prev_ref,
                      out_ref):
    del prev_ref
    _mlp_body(flat_ref, w1_ref, b1_ref, w2t_ref, b2_ref, out_ref)


def _mlp_chunk(flat, W1, b1, W2T, b2, prev, cb, row0):
    base = row0 // BM
    in_specs = [
        pl.BlockSpec((BM * SEQ, EMBED), lambda i: (i, 0)),
        pl.BlockSpec((SEQ * EMBED, HIDDEN), lambda i: (0, 0)),
        pl.BlockSpec((1, HIDDEN), lambda i: (0, 0)),
        pl.BlockSpec((OUT, HIDDEN), lambda i: (0, 0)),
        pl.BlockSpec((OUT, 1), lambda i: (0, 0)),
    ]
    args = [flat, W1, b1, W2T, b2]
    body = _mlp_body
    aliases = {}
    if prev is not None:
        in_specs.append(pl.BlockSpec(memory_space=pl.ANY))
        args.append(prev)
        body = _mlp_body_aliased
        aliases = {5: 0}
    return pl.pallas_call(
        body,
        grid=(cb // BM,),
        in_specs=in_specs,
        out_specs=pl.BlockSpec((OUT, BM), lambda i: (0, base + i)),
        out_shape=jax.ShapeDtypeStruct((OUT, BATCH), jnp.float32),
        input_output_aliases=aliases,
    )(*args)


def kernel(x, table, W1, b1, W2, b2):
    w1_h = W1.astype(jnp.bfloat16)
    w2t_h = W2.T.astype(jnp.bfloat16)
    b1r = b1.reshape(1, HIDDEN)
    b2r = b2.reshape(OUT, 1)
    starts = [sum(CHUNKS[:c]) for c in range(len(CHUNKS))]
    idx_all = x.reshape(1, BATCH * SEQ)
    flats = [_sc_gather(table, idx_all, cb * SEQ, r0 * SEQ)
             for r0, cb in zip(starts, CHUNKS)]
    out_t = None
    for c, (r0, cb) in enumerate(zip(starts, CHUNKS)):
        out_t = _mlp_chunk(flats[c], w1_h, b1r, w2t_h, b2r, out_t, cb, r0)
    return out_t.T
